# baseline (speedup 1.0000x reference)
import functools

import jax
import jax.numpy as jnp
from jax import lax
from jax.experimental import pallas as pl
from jax.experimental.pallas import tpu as pltpu
from jax.experimental.pallas import tpu_sc as plsc

N = 10000
FIN = 128
FOUT = 32
K = 4
NNZ = N * 32

NC = 2
NS = 16
CHUNK = 128
SS = 5
FAST_CORE = 0
NCH = 160
NSS = NCH // SS
EDGES_PAD = NS * NCH * CHUNK
NPAD = 10240
ROWS_PER_TILE = NPAD // NS
HALF = ROWS_PER_TILE // 2

_LANE = 16
_SSE = SS * CHUNK


def _splat(vv, l):
    idx = jnp.full((_LANE, 1), l, jnp.int32)
    dn = lax.GatherDimensionNumbers(
        offset_dims=(), collapsed_slice_dims=(0,), start_index_map=(0,))
    return lax.gather(vv, idx, dn, slice_sizes=(1,),
                      mode=lax.GatherScatterMode.PROMISE_IN_BOUNDS)


@functools.partial(
    pl.kernel,
    out_type=jax.ShapeDtypeStruct((NPAD, FOUT), jnp.float32),
    mesh=plsc.VectorSubcoreMesh(core_axis_name="c", subcore_axis_name="s"),
    scratch_types=[
        pltpu.VMEM((NCH, CHUNK), jnp.int32),
        pltpu.VMEM((NCH, CHUNK), jnp.int32),
        pltpu.VMEM((NCH * 8, _LANE), jnp.float32),
        pltpu.VMEM((2, _SSE, FOUT), jnp.float32),
        pltpu.VMEM((_LANE,), jnp.float32),
        pltpu.VMEM_SHARED((NPAD, FOUT), jnp.float32),
        pltpu.SemaphoreType.DMA,
        pltpu.SemaphoreType.DMA,
        pltpu.SemaphoreType.DMA,
        pltpu.SemaphoreType.DMA,
    ],
    compiler_params=pltpu.CompilerParams(use_tc_tiling_on_sc=False),
)
def _step_sc(b_hbm, u_hbm, c_hbm, coef_hbm, cols_all, rows_all, vals_all,
             out_hbm, colv, rowv, valv, gbuf, coefv, acc,
             sg0, sg1, ss0, ss1):
    c = lax.axis_index("c")
    s = lax.axis_index("s")
    r0 = s * ROWS_PER_TILE
    sem_g = (sg0, sg1)
    sem_s = (ss0, ss1)

    @pl.when(c == FAST_CORE)
    def _():
        zero16 = jnp.zeros((_LANE,), jnp.float32)

        def zrow(i, _):
            gbuf[0, i, pl.ds(0, _LANE)] = zero16
            gbuf[0, i, pl.ds(_LANE, _LANE)] = zero16
            return 0

        lax.fori_loop(0, ROWS_PER_TILE, zrow, 0)
        pltpu.sync_copy(gbuf.at[0, pl.ds(0, ROWS_PER_TILE)],
                        acc.at[pl.ds(r0, ROWS_PER_TILE)])
        pltpu.sync_copy(cols_all.at[pl.ds(s * NCH, NCH)], colv)
        pltpu.sync_copy(rows_all.at[pl.ds(s * NCH, NCH)], rowv)
        pltpu.sync_copy(vals_all.at[pl.ds(s * NCH * 8, NCH * 8)], valv)
        pltpu.sync_copy(coef_hbm, coefv)
        plsc.subcore_barrier()

        def issue_gathers(t, bi):
            for b in range(SS):
                pltpu.async_copy(b_hbm.at[colv.at[t * SS + b]],
                                 gbuf.at[bi, pl.ds(b * CHUNK, CHUNK)],
                                 sem_g[bi])

        def drain_gathers(bi):
            pltpu.make_async_copy(b_hbm.at[pl.ds(0, _SSE)],
                                  gbuf.at[bi], sem_g[bi]).wait()

        def drain_scatters(bi):
            pltpu.make_async_copy(gbuf.at[bi], acc.at[pl.ds(0, _SSE)],
                                  sem_s[bi]).wait()

        def compute_and_scatter(t, bi):
            for b in range(SS):
                def grp(g, _, b=b):
                    vv = valv[(t * SS + b) * (CHUNK // _LANE) + g]
                    for l in range(_LANE):
                        sp = _splat(vv, l)
                        e = b * CHUNK + g * _LANE + l
                        gbuf[bi, e, pl.ds(0, _LANE)] = (
                            gbuf[bi, e, pl.ds(0, _LANE)] * sp)
                        gbuf[bi, e, pl.ds(_LANE, _LANE)] = (
                            gbuf[bi, e, pl.ds(_LANE, _LANE)] * sp)
                    return 0
                lax.fori_loop(0, 8, grp, 0)
                pltpu.async_copy(gbuf.at[bi, pl.ds(b * CHUNK, CHUNK)],
                                 acc.at[rowv.at[t * SS + b]], sem_s[bi],
                                 add=True)

        issue_gathers(0, 0)
        issue_gathers(1, 1)
        drain_gathers(0)
        compute_and_scatter(0, 0)

        def pair(tt, carry):
            t_odd = 2 * tt + 1
            drain_scatters(0)
            issue_gathers(t_odd + 1, 0)
            drain_gathers(1)
            compute_and_scatter(t_odd, 1)
            drain_scatters(1)
            issue_gathers(t_odd + 2, 1)
            drain_gathers(0)
            compute_and_scatter(t_odd + 1, 0)
            return carry

        lax.fori_loop(0, (NSS - 2) // 2, pair, 0)
        drain_scatters(0)
        drain_gathers(1)
        compute_and_scatter(NSS - 1, 1)
        drain_scatters(1)

        plsc.subcore_barrier()

        al = _splat(coefv[pl.ds(0, _LANE)], 0)
        ga = _splat(coefv[pl.ds(0, _LANE)], 1)
        sel = _splat(coefv[pl.ds(0, _LANE)], 2)

        for h in range(2):
            rh = r0 + h * HALF
            pltpu.sync_copy(acc.at[pl.ds(rh, HALF)],
                            gbuf.at[0, pl.ds(0, HALF)])
            pltpu.sync_copy(u_hbm.at[pl.ds(rh, HALF)],
                            gbuf.at[0, pl.ds(HALF, HALF)])
            pltpu.sync_copy(c_hbm.at[pl.ds(rh, HALF)],
                            gbuf.at[1, pl.ds(0, HALF)])

            def crow(i, _):
                for p in range(2):
                    d = pl.ds(p * _LANE, _LANE)
                    a = gbuf[0, i, d]
                    uu = gbuf[0, HALF + i, d]
                    cp = gbuf[1, i, d]
                    r = al * a + uu - ga * cp
                    gbuf[0, i, d] = jnp.maximum(r, r * sel)
                return 0

            lax.fori_loop(0, HALF, crow, 0)
            pltpu.sync_copy(gbuf.at[0, pl.ds(0, HALF)],
                            out_hbm.at[pl.ds(rh, HALF)])


def _mm_body(x_ref, w_ref, o_ref):
    o_ref[...] = jnp.dot(x_ref[...], w_ref[...],
                         preferred_element_type=jnp.float32)


def _theta_matmul(x, w):
    blk = 2000
    return pl.pallas_call(
        _mm_body,
        grid=(N // blk,),
        in_specs=[pl.BlockSpec((blk, FIN), lambda i: (i, 0)),
                  pl.BlockSpec((FIN, K * FOUT), lambda i: (0, 0))],
        out_specs=pl.BlockSpec((blk, K * FOUT), lambda i: (i, 0)),
        out_shape=jax.ShapeDtypeStruct((N, K * FOUT), jnp.float32),
    )(x, w)


def kernel(x, lap_indices, lap_values, theta):
    pad = EDGES_PAD - NNZ
    rows = jnp.concatenate([lap_indices[0], jnp.zeros((pad,), jnp.int32)])
    cols = jnp.concatenate([lap_indices[1], jnp.zeros((pad,), jnp.int32)])
    vals = jnp.concatenate([lap_values, jnp.zeros((pad,), jnp.float32)])
    rows_a = rows.reshape(EDGES_PAD // CHUNK, CHUNK)
    cols_a = cols.reshape(EDGES_PAD // CHUNK, CHUNK)
    vals_a = vals.reshape(EDGES_PAD // _LANE, _LANE)

    w = jnp.transpose(theta, (1, 0, 2)).reshape(FIN, K * FOUT)
    big_u = _theta_matmul(x, w)
    rpad = ((0, NPAD - N), (0, 0))
    u = [jnp.pad(big_u[:, k * FOUT:(k + 1) * FOUT], rpad) for k in range(K)]

    u_scan = jnp.stack([u[k] for k in range(K - 2, -1, -1)])
    coef_all = jnp.zeros((K - 1, _LANE), jnp.float32)
    coef_all = coef_all.at[:, 0].set(jnp.array([2.0] * (K - 2) + [1.0]))
    coef_all = coef_all.at[:, 1].set(jnp.array([0.0] + [1.0] * (K - 2)))
    coef_all = coef_all.at[:, 2].set(jnp.array([1.0] * (K - 2) + [0.0]))
    nsteps = (K - 1) + (lap_values[0] * 0.0).astype(jnp.int32)

    def cond(st):
        return st[0] < nsteps

    def step(st):
        i, bk1, bk2 = st
        uk = lax.dynamic_index_in_dim(u_scan, i, 0, keepdims=False)
        coef = lax.dynamic_index_in_dim(coef_all, i, 0, keepdims=False)
        bk = _step_sc(bk1, uk, bk2, coef, cols_a, rows_a, vals_a)
        return (i + 1, bk, bk1)

    init = (jnp.int32(0), u[K - 1], jnp.zeros((NPAD, FOUT), jnp.float32))
    _, sfin, _ = lax.while_loop(cond, step, init)
    return sfin[:N]


# speedup vs baseline: 1.2980x; 1.2980x over previous
import functools

import jax
import jax.numpy as jnp
from jax import lax
from jax.experimental import pallas as pl
from jax.experimental.pallas import tpu as pltpu
from jax.experimental.pallas import tpu_sc as plsc

N = 10000
FIN = 128
FOUT = 32
K = 4
NNZ = N * 32

NC = 2
NS = 16
CHUNK = 128
SS = 5
FAST_CORE = 0
NCH = 160
NSS = NCH // SS
EDGES_PAD = NS * NCH * CHUNK
NPAD = 10240
ROWS_PER_TILE = NPAD // NS
HALF = ROWS_PER_TILE // 2

_LANE = 16
_SSE = SS * CHUNK


def _splat(vv, l):
    idx = jnp.full((_LANE, 1), l, jnp.int32)
    dn = lax.GatherDimensionNumbers(
        offset_dims=(), collapsed_slice_dims=(0,), start_index_map=(0,))
    return lax.gather(vv, idx, dn, slice_sizes=(1,),
                      mode=lax.GatherScatterMode.PROMISE_IN_BOUNDS)


@functools.partial(
    pl.kernel,
    out_type=jax.ShapeDtypeStruct((NPAD, FOUT), jnp.float32),
    mesh=plsc.VectorSubcoreMesh(core_axis_name="c", subcore_axis_name="s"),
    scratch_types=[
        pltpu.VMEM((NCH, CHUNK), jnp.int32),
        pltpu.VMEM((NCH, CHUNK), jnp.int32),
        pltpu.VMEM((NCH * 8, _LANE), jnp.float32),
        pltpu.VMEM((2, _SSE, FOUT), jnp.float32),
        pltpu.VMEM((_LANE,), jnp.float32),
        pltpu.VMEM_SHARED((NPAD, FOUT), jnp.float32),
        pltpu.SemaphoreType.DMA,
        pltpu.SemaphoreType.DMA,
        pltpu.SemaphoreType.DMA,
        pltpu.SemaphoreType.DMA,
    ],
    compiler_params=pltpu.CompilerParams(use_tc_tiling_on_sc=False),
)
def _step_sc(b_hbm, cols_all, rows_all, vals_all,
             out_hbm, colv, rowv, valv, gbuf, coefv, acc,
             sg0, sg1, ss0, ss1):
    c = lax.axis_index("c")
    s = lax.axis_index("s")
    r0 = s * ROWS_PER_TILE
    sem_g = (sg0, sg1)
    sem_s = (ss0, ss1)

    @pl.when(c == FAST_CORE)
    def _():
        zero16 = jnp.zeros((_LANE,), jnp.float32)

        def zrow(i, _):
            gbuf[0, i, pl.ds(0, _LANE)] = zero16
            gbuf[0, i, pl.ds(_LANE, _LANE)] = zero16
            return 0

        lax.fori_loop(0, ROWS_PER_TILE, zrow, 0)
        pltpu.sync_copy(gbuf.at[0, pl.ds(0, ROWS_PER_TILE)],
                        acc.at[pl.ds(r0, ROWS_PER_TILE)])
        pltpu.sync_copy(cols_all.at[pl.ds(s * NCH, NCH)], colv)
        pltpu.sync_copy(rows_all.at[pl.ds(s * NCH, NCH)], rowv)
        pltpu.sync_copy(vals_all.at[pl.ds(s * NCH * 8, NCH * 8)], valv)
        plsc.subcore_barrier()

        def issue_gathers(t, bi):
            for b in range(SS):
                pltpu.async_copy(b_hbm.at[colv.at[t * SS + b]],
                                 gbuf.at[bi, pl.ds(b * CHUNK, CHUNK)],
                                 sem_g[bi])

        def drain_gathers(bi):
            pltpu.make_async_copy(b_hbm.at[pl.ds(0, _SSE)],
                                  gbuf.at[bi], sem_g[bi]).wait()

        def drain_scatters(bi):
            pltpu.make_async_copy(gbuf.at[bi], acc.at[pl.ds(0, _SSE)],
                                  sem_s[bi]).wait()

        def compute_and_scatter(t, bi):
            for b in range(SS):
                def grp(g, _, b=b):
                    vv = valv[(t * SS + b) * (CHUNK // _LANE) + g]
                    for l in range(_LANE):
                        sp = _splat(vv, l)
                        e = b * CHUNK + g * _LANE + l
                        gbuf[bi, e, pl.ds(0, _LANE)] = (
                            gbuf[bi, e, pl.ds(0, _LANE)] * sp)
                        gbuf[bi, e, pl.ds(_LANE, _LANE)] = (
                            gbuf[bi, e, pl.ds(_LANE, _LANE)] * sp)
                    return 0
                lax.fori_loop(0, 8, grp, 0)
                pltpu.async_copy(gbuf.at[bi, pl.ds(b * CHUNK, CHUNK)],
                                 acc.at[rowv.at[t * SS + b]], sem_s[bi],
                                 add=True)

        issue_gathers(0, 0)
        issue_gathers(1, 1)
        drain_gathers(0)
        compute_and_scatter(0, 0)

        def pair(tt, carry):
            t_odd = 2 * tt + 1
            drain_scatters(0)
            issue_gathers(t_odd + 1, 0)
            drain_gathers(1)
            compute_and_scatter(t_odd, 1)
            drain_scatters(1)
            issue_gathers(t_odd + 2, 1)
            drain_gathers(0)
            compute_and_scatter(t_odd + 1, 0)
            return carry

        lax.fori_loop(0, (NSS - 2) // 2, pair, 0)
        drain_scatters(0)
        drain_gathers(1)
        compute_and_scatter(NSS - 1, 1)
        drain_scatters(1)

        plsc.subcore_barrier()
        pltpu.sync_copy(acc.at[pl.ds(r0, ROWS_PER_TILE)],
                        out_hbm.at[pl.ds(r0, ROWS_PER_TILE)])


def _mm_body(x_ref, w_ref, o_ref):
    o_ref[...] = jnp.dot(x_ref[...], w_ref[...],
                         preferred_element_type=jnp.float32)


def _theta_matmul(x, w):
    blk = 2000
    return pl.pallas_call(
        _mm_body,
        grid=(N // blk,),
        in_specs=[pl.BlockSpec((blk, FIN), lambda i: (i, 0)),
                  pl.BlockSpec((FIN, K * FOUT), lambda i: (0, 0))],
        out_specs=pl.BlockSpec((blk, K * FOUT), lambda i: (i, 0)),
        out_shape=jax.ShapeDtypeStruct((N, K * FOUT), jnp.float32),
    )(x, w)


_FLAT = (NPAD * FOUT // FIN, FIN)


def _comb_body(p, u, cm, al, ga, o):
    o[...] = al[0, 0] * p[...] + u[...] - ga[0, 0] * cm[...]


def _combine(p, u, cm, alpha, gamma):
    out = pl.pallas_call(
        _comb_body,
        out_shape=jax.ShapeDtypeStruct(_FLAT, jnp.float32),
    )(p.reshape(_FLAT), u.reshape(_FLAT), cm.reshape(_FLAT),
      alpha.reshape(1, 1), gamma.reshape(1, 1))
    return out.reshape(NPAD, FOUT)


def _relu_body(x, o):
    o[...] = jnp.maximum(x[...], 0.0)


def _relu(x):
    out = pl.pallas_call(
        _relu_body,
        out_shape=jax.ShapeDtypeStruct(_FLAT, jnp.float32),
    )(x.reshape(_FLAT))
    return out.reshape(NPAD, FOUT)


def kernel(x, lap_indices, lap_values, theta):
    pad = EDGES_PAD - NNZ
    rows = jnp.concatenate([lap_indices[0], jnp.zeros((pad,), jnp.int32)])
    cols = jnp.concatenate([lap_indices[1], jnp.zeros((pad,), jnp.int32)])
    vals = jnp.concatenate([lap_values, jnp.zeros((pad,), jnp.float32)])
    rows_a = rows.reshape(EDGES_PAD // CHUNK, CHUNK)
    cols_a = cols.reshape(EDGES_PAD // CHUNK, CHUNK)
    vals_a = vals.reshape(EDGES_PAD // _LANE, _LANE)

    w = jnp.transpose(theta, (1, 0, 2)).reshape(FIN, K * FOUT)
    big_u = _theta_matmul(x, w)
    rpad = ((0, NPAD - N), (0, 0))
    u = [jnp.pad(big_u[:, k * FOUT:(k + 1) * FOUT], rpad) for k in range(K)]

    u_scan = jnp.stack([u[k] for k in range(K - 2, -1, -1)])
    alphas = jnp.array([2.0] * (K - 2) + [1.0], jnp.float32)
    gammas = jnp.array([0.0] + [1.0] * (K - 2), jnp.float32)
    nsteps = (K - 1) + (lap_values[0] * 0.0).astype(jnp.int32)

    def cond(st):
        return st[0] < nsteps

    def step(st):
        i, bk1, bk2 = st
        uk = lax.dynamic_index_in_dim(u_scan, i, 0, keepdims=False)
        al = lax.dynamic_index_in_dim(alphas, i, 0, keepdims=False)
        ga = lax.dynamic_index_in_dim(gammas, i, 0, keepdims=False)
        p = _step_sc(bk1, cols_a, rows_a, vals_a)
        bk = _combine(p, uk, bk2, al, ga)
        return (i + 1, bk, bk1)

    init = (jnp.int32(0), u[K - 1], jnp.zeros((NPAD, FOUT), jnp.float32))
    _, sfin, _ = lax.while_loop(cond, step, init)
    return _relu(sfin)[:N]
